# baseline (speedup 1.0000x reference)
import functools

import jax
import jax.numpy as jnp
from jax import lax
from jax.experimental import pallas as pl
from jax.experimental.pallas import tpu as pltpu
from jax.experimental.pallas import tpu_sc as plsc

N = 50000
NB = 50000
E = 800000
R = 1000
D_FEAT = 100
D_REL = 300
D_HID = 200

C = 16
K = 13
DP = K * C
DEF = 112
NTAB = 50048
NBQ = 53248
EP = 802816
DUMMY = 50000
B = 128

STRIPE = NTAB // 16
RPT32 = NBQ // 32
RPT16 = NBQ // 16
EPT = EP // 16

_MESH = plsc.VectorSubcoreMesh(core_axis_name="c", subcore_axis_name="s")
_SC_PARAMS = pltpu.CompilerParams(use_tc_tiling_on_sc=False)
_RB = 400
_GRID = NB // _RB


def _t0_body(rel_ref, wr_ref, br_ref, w1b_ref, out_ref):
    rc = jnp.dot(rel_ref[...], wr_ref[...], preferred_element_type=jnp.float32)
    rc = jnp.maximum(rc + br_ref[...], 0.0)
    out_ref[...] = jnp.dot(rc, w1b_ref[...], preferred_element_type=jnp.float32)


def _t0(rel, wr, br, w1b):
    return pl.pallas_call(
        _t0_body,
        out_shape=jax.ShapeDtypeStruct((R, DP), jnp.float32),
    )(rel, wr, br, w1b)


def _t2_body(ef_ref, s1b_ref, w_ref, out_ref):
    s = jnp.dot(ef_ref[...], w_ref[...], preferred_element_type=jnp.float32)
    s = s + s1b_ref[...]
    for k in range(K):
        out_ref[k] = s[:, k * C:(k + 1) * C]


def _t2(ef, s1b, w):
    return pl.pallas_call(
        _t2_body,
        grid=(_GRID,),
        in_specs=[
            pl.BlockSpec((_RB, DEF), lambda i: (i, 0)),
            pl.BlockSpec((_RB, DP), lambda i: (i, 0)),
            pl.BlockSpec((DEF, DP), lambda i: (0, 0)),
        ],
        out_specs=pl.BlockSpec((K, _RB, C), lambda i: (0, i, 0)),
        out_shape=jax.ShapeDtypeStruct((K, NTAB, C), jnp.float32),
    )(ef, s1b, w)


def _t4_body(agg_ref, sup_ref, deg_ref, w_ref, out_ref):
    inv = 1.0 / (deg_ref[:, :1] + 1.0)
    hs = [(agg_ref[k] + sup_ref[k]) * inv for k in range(K)]
    h = jnp.maximum(jnp.concatenate(hs, axis=1), 0.0)
    s2 = jnp.dot(h, w_ref[...], preferred_element_type=jnp.float32)
    for k in range(K):
        out_ref[k] = s2[:, k * C:(k + 1) * C]


def _t4(agg, sup, deg, w):
    return pl.pallas_call(
        _t4_body,
        grid=(_GRID,),
        in_specs=[
            pl.BlockSpec((K, _RB, C), lambda i: (0, i, 0)),
            pl.BlockSpec((K, _RB, C), lambda i: (0, i, 0)),
            pl.BlockSpec((_RB, C), lambda i: (i, 0)),
            pl.BlockSpec((DP, DP), lambda i: (0, 0)),
        ],
        out_specs=pl.BlockSpec((K, _RB, C), lambda i: (0, i, 0)),
        out_shape=jax.ShapeDtypeStruct((K, NTAB, C), jnp.float32),
    )(agg, sup, deg, w)


def _t6_body(agg_ref, sup_ref, deg_ref, out_ref):
    inv = 1.0 / (deg_ref[:, :1] + 1.0)
    for k in range(K):
        out_ref[k] = (agg_ref[k] + sup_ref[k]) * inv


def _t6(agg, sup, deg):
    return pl.pallas_call(
        _t6_body,
        grid=(_GRID,),
        in_specs=[
            pl.BlockSpec((K, _RB, C), lambda i: (0, i, 0)),
            pl.BlockSpec((K, _RB, C), lambda i: (0, i, 0)),
            pl.BlockSpec((_RB, C), lambda i: (i, 0)),
        ],
        out_specs=pl.BlockSpec((K, _RB, C), lambda i: (0, i, 0)),
        out_shape=jax.ShapeDtypeStruct((K, NBQ, C), jnp.float32),
    )(agg, sup, deg)


def _t8_body(sums_ref, cnt_ref, out_ref):
    inv = 1.0 / jnp.maximum(cnt_ref[:, :1], 1.0)
    vals = [sums_ref[k] * inv for k in range(K)]
    out_ref[...] = jnp.concatenate(vals, axis=1)


def _t8(sums, cnt):
    return pl.pallas_call(
        _t8_body,
        grid=(_GRID,),
        in_specs=[
            pl.BlockSpec((K, _RB, C), lambda i: (0, i, 0)),
            pl.BlockSpec((_RB, C), lambda i: (i, 0)),
        ],
        out_specs=pl.BlockSpec((_RB, DP), lambda i: (i, 0)),
        out_shape=jax.ShapeDtypeStruct((NTAB, DP), jnp.float32),
    )(sums, cnt)


@functools.partial(
    pl.kernel,
    compiler_params=_SC_PARAMS,
    out_type=(jax.ShapeDtypeStruct((NBQ, DEF), jnp.float32),
              jax.ShapeDtypeStruct((NBQ, DP), jnp.float32)),
    mesh=_MESH,
    scratch_types=[
        pltpu.VMEM((B,), jnp.int32),
        pltpu.VMEM((B,), jnp.int32),
        pltpu.VMEM((B, DEF), jnp.float32),
        pltpu.VMEM((B, DP), jnp.float32),
        pltpu.SemaphoreType.DMA,
        pltpu.SemaphoreType.DMA,
    ],
)
def _s1(bx_hbm, bgi_hbm, ef_hbm, rcw_hbm, ef_out, s1b_out,
        idx1, idx2, rows1, rows2, sem1, sem2):
    wid = lax.axis_index("s") * 2 + lax.axis_index("c")
    base0 = wid * RPT32

    @pl.loop(0, RPT32, step=B)
    def _(off):
        base = base0 + off
        pltpu.sync_copy(bx_hbm.at[pl.ds(base, B)], idx1)
        pltpu.sync_copy(bgi_hbm.at[pl.ds(base, B)], idx2)
        c1 = pltpu.async_copy(ef_hbm.at[idx1], rows1, sem1)
        c2 = pltpu.async_copy(rcw_hbm.at[idx2], rows2, sem2)
        c1.wait()
        c2.wait()
        pltpu.sync_copy(rows1, ef_out.at[pl.ds(base, B)])
        pltpu.sync_copy(rows2, s1b_out.at[pl.ds(base, B)])


@functools.partial(
    pl.kernel,
    compiler_params=_SC_PARAMS,
    out_type=(jax.ShapeDtypeStruct((NTAB, C), jnp.float32),
              jax.ShapeDtypeStruct((NTAB, C), jnp.float32)),
    mesh=_MESH,
    scratch_types=[
        pltpu.VMEM((B,), jnp.int32),
        pltpu.VMEM((B, C), jnp.float32),
        pltpu.VMEM((STRIPE, C), jnp.float32),
        pltpu.VMEM_SHARED((NTAB, C), jnp.float32),
    ],
)
def _s3a(dst_hbm, bx_hbm, ones_hbm, zeros_hbm, deg_out, cnt_out,
         didx, ones_v, zstripe, acc):
    cid = lax.axis_index("c")
    sid = lax.axis_index("s")
    pltpu.sync_copy(zeros_hbm, zstripe)
    pltpu.sync_copy(ones_hbm, ones_v)
    pltpu.sync_copy(zstripe, acc.at[pl.ds(sid * STRIPE, STRIPE)])
    plsc.subcore_barrier()

    @pl.when(cid == 0)
    def _():
        @pl.loop(0, EPT, step=B)
        def _(off):
            pltpu.sync_copy(dst_hbm.at[pl.ds(sid * EPT + off, B)], didx)
            pltpu.sync_copy(ones_v, acc.at[didx], add=True)

    @pl.when(cid == 1)
    def _():
        @pl.loop(0, RPT16, step=B)
        def _(off):
            pltpu.sync_copy(bx_hbm.at[pl.ds(sid * RPT16 + off, B)], didx)
            pltpu.sync_copy(ones_v, acc.at[didx], add=True)

    plsc.subcore_barrier()
    stripe_slc = pl.ds(sid * STRIPE, STRIPE)

    @pl.when(cid == 0)
    def _():
        pltpu.sync_copy(acc.at[stripe_slc], deg_out.at[stripe_slc])

    @pl.when(cid == 1)
    def _():
        pltpu.sync_copy(acc.at[stripe_slc], cnt_out.at[stripe_slc])


@functools.partial(
    pl.kernel,
    compiler_params=_SC_PARAMS,
    out_type=jax.ShapeDtypeStruct((K, NTAB, C), jnp.float32),
    mesh=_MESH,
    scratch_types=[
        pltpu.VMEM((B,), jnp.int32),
        pltpu.VMEM((B,), jnp.int32),
        pltpu.VMEM((B, C), jnp.float32),
        pltpu.VMEM((STRIPE, C), jnp.float32),
        pltpu.VMEM_SHARED((NTAB, C), jnp.float32),
        pltpu.SemaphoreType.DMA,
    ],
)
def _s3(src_hbm, dst_hbm, sup_hbm, zeros_hbm, agg_out,
        sidx, didx, rows, zstripe, acc, sem):
    cid = lax.axis_index("c")
    sid = lax.axis_index("s")
    pltpu.sync_copy(zeros_hbm, zstripe)
    ebase0 = sid * EPT
    stripe_slc = pl.ds(sid * STRIPE, STRIPE)
    for k in range(K):
        @pl.when((k % 2) == cid)
        def _():
            pltpu.sync_copy(zstripe, acc.at[stripe_slc])
            plsc.subcore_barrier()

            @pl.loop(0, EPT, step=B)
            def _(off):
                base = ebase0 + off
                pltpu.sync_copy(src_hbm.at[pl.ds(base, B)], sidx)
                pltpu.sync_copy(dst_hbm.at[pl.ds(base, B)], didx)
                pltpu.async_copy(sup_hbm.at[k].at[sidx], rows, sem).wait()
                pltpu.sync_copy(rows, acc.at[didx], add=True)

            plsc.subcore_barrier()
            pltpu.sync_copy(acc.at[stripe_slc], agg_out.at[k].at[stripe_slc])
            plsc.subcore_barrier()


@functools.partial(
    pl.kernel,
    compiler_params=_SC_PARAMS,
    out_type=jax.ShapeDtypeStruct((K, NTAB, C), jnp.float32),
    mesh=_MESH,
    scratch_types=[
        pltpu.VMEM((B,), jnp.int32),
        pltpu.VMEM((B, C), jnp.float32),
        pltpu.VMEM((STRIPE, C), jnp.float32),
        pltpu.VMEM_SHARED((NTAB, C), jnp.float32),
    ],
)
def _s7(bx_hbm, e_hbm, zeros_hbm, sums_out, didx, rows, zstripe, acc):
    cid = lax.axis_index("c")
    sid = lax.axis_index("s")
    pltpu.sync_copy(zeros_hbm, zstripe)
    rbase0 = sid * RPT16
    stripe_slc = pl.ds(sid * STRIPE, STRIPE)
    for k in range(K):
        @pl.when((k % 2) == cid)
        def _():
            pltpu.sync_copy(zstripe, acc.at[stripe_slc])
            plsc.subcore_barrier()

            @pl.loop(0, RPT16, step=B)
            def _(off):
                base = rbase0 + off
                pltpu.sync_copy(bx_hbm.at[pl.ds(base, B)], didx)
                pltpu.sync_copy(e_hbm.at[k].at[pl.ds(base, B)], rows)
                pltpu.sync_copy(rows, acc.at[didx], add=True)

            plsc.subcore_barrier()
            pltpu.sync_copy(acc.at[stripe_slc], sums_out.at[k].at[stripe_slc])
            plsc.subcore_barrier()


@functools.partial(
    pl.kernel,
    compiler_params=_SC_PARAMS,
    out_type=jax.ShapeDtypeStruct((NBQ, DP), jnp.float32),
    mesh=_MESH,
    scratch_types=[
        pltpu.VMEM((B,), jnp.int32),
        pltpu.VMEM((B, DP), jnp.float32),
        pltpu.SemaphoreType.DMA,
    ],
)
def _s9(bx_hbm, tab_hbm, z_out, idx, rows, sem):
    wid = lax.axis_index("s") * 2 + lax.axis_index("c")
    base0 = wid * RPT32

    @pl.loop(0, RPT32, step=B)
    def _(off):
        base = base0 + off
        pltpu.sync_copy(bx_hbm.at[pl.ds(base, B)], idx)
        pltpu.async_copy(tab_hbm.at[idx], rows, sem).wait()
        pltpu.sync_copy(rows, z_out.at[pl.ds(base, B)])


def kernel(entity_feat, relation_embeddings, W_rel_in, b_rel_in,
           W_gcn1, W_gcn2, b_x, b_node_graph_index, edge_index):
    f32 = jnp.float32
    efp = jnp.pad(entity_feat, ((0, NTAB - N), (0, DEF - D_FEAT)))
    w1p = jnp.pad(W_gcn1, ((0, 0), (0, DP - D_HID)))
    w1top = jnp.pad(w1p[:D_FEAT], ((0, DEF - D_FEAT), (0, 0)))
    w1bot = w1p[D_FEAT:D_HID]
    w2p = jnp.pad(W_gcn2, ((0, DP - D_HID), (0, DP - D_HID)))
    pad_bx = jnp.full((NBQ - NB,), DUMMY, jnp.int32)
    bxp = jnp.concatenate([b_x.astype(jnp.int32), pad_bx])
    bgip = jnp.concatenate([b_node_graph_index.astype(jnp.int32),
                            jnp.zeros((NBQ - NB,), jnp.int32)])
    pad_e = jnp.full((EP - E,), DUMMY, jnp.int32)
    srcp = jnp.concatenate([edge_index[0].astype(jnp.int32), pad_e])
    dstp = jnp.concatenate([edge_index[1].astype(jnp.int32), pad_e])
    zeros_hbm = jnp.zeros((STRIPE, C), f32)
    ones_hbm = jnp.ones((B, C), f32)
    brel = b_rel_in.reshape(1, D_FEAT)

    rcw = _t0(relation_embeddings, W_rel_in, brel, w1bot)
    ef_g, s1b = _s1(bxp, bgip, efp, rcw)
    deg, cnt = _s3a(dstp, bxp, ones_hbm, zeros_hbm)
    sup1 = _t2(ef_g, s1b, w1top)
    agg1 = _s3(srcp, dstp, sup1, zeros_hbm)
    sup2 = _t4(agg1, sup1, deg, w2p)
    agg2 = _s3(srcp, dstp, sup2, zeros_hbm)
    e = _t6(agg2, sup2, deg)
    sums = _s7(bxp, e, zeros_hbm)
    out_tab = _t8(sums, cnt)
    zf = _s9(bxp, out_tab)
    return zf[:NB, :D_HID]


# speedup vs baseline: 2.6832x; 2.6832x over previous
import functools

import jax
import jax.numpy as jnp
from jax import lax
from jax.experimental import pallas as pl
from jax.experimental.pallas import tpu as pltpu
from jax.experimental.pallas import tpu_sc as plsc

N = 50000
NB = 50000
E = 800000
R = 1000
D_FEAT = 100
D_REL = 300
D_HID = 200

C = 32
K = 8
DP = K * C
DEF = 112
NTAB = 50048
NBQ = 53248
EP = 802816
DUMMY = 50000
B = 128
SUP = 2
NSUP = (EP // 16) // (SUP * B)

STRIPE = NTAB // 16
RPT32 = NBQ // 32
RPT16 = NBQ // 16
EBT = EP // (16 * B)

_MESH = plsc.VectorSubcoreMesh(core_axis_name="c", subcore_axis_name="s")
_SC_PARAMS = pltpu.CompilerParams(use_tc_tiling_on_sc=False)
_RB = 400
_GRID = NB // _RB


def _t0_body(rel_ref, wr_ref, br_ref, w1b_ref, out_ref):
    rc = jnp.dot(rel_ref[...], wr_ref[...], preferred_element_type=jnp.float32)
    rc = jnp.maximum(rc + br_ref[...], 0.0)
    out_ref[...] = jnp.dot(rc, w1b_ref[...], preferred_element_type=jnp.float32)


def _t0(rel, wr, br, w1b):
    return pl.pallas_call(
        _t0_body,
        out_shape=jax.ShapeDtypeStruct((R, DP), jnp.float32),
    )(rel, wr, br, w1b)


def _t2_body(ef_ref, s1b_ref, w_ref, out_ref):
    s = jnp.dot(ef_ref[...], w_ref[...], preferred_element_type=jnp.float32)
    s = s + s1b_ref[...]
    for k in range(K):
        out_ref[k] = s[:, k * C:(k + 1) * C]


def _t2(ef, s1b, w):
    return pl.pallas_call(
        _t2_body,
        grid=(_GRID,),
        in_specs=[
            pl.BlockSpec((_RB, DEF), lambda i: (i, 0)),
            pl.BlockSpec((_RB, DP), lambda i: (i, 0)),
            pl.BlockSpec((DEF, DP), lambda i: (0, 0)),
        ],
        out_specs=pl.BlockSpec((K, _RB, C), lambda i: (0, i, 0)),
        out_shape=jax.ShapeDtypeStruct((K, NTAB, C), jnp.float32),
    )(ef, s1b, w)


def _t4_body(agg_ref, sup_ref, deg_ref, w_ref, out_ref):
    inv = 1.0 / (deg_ref[:, :1] + 1.0)
    hs = [(agg_ref[k] + sup_ref[k]) * inv for k in range(K)]
    h = jnp.maximum(jnp.concatenate(hs, axis=1), 0.0)
    s2 = jnp.dot(h, w_ref[...], preferred_element_type=jnp.float32)
    for k in range(K):
        out_ref[k] = s2[:, k * C:(k + 1) * C]


def _t4(agg, sup, deg, w):
    return pl.pallas_call(
        _t4_body,
        grid=(_GRID,),
        in_specs=[
            pl.BlockSpec((K, _RB, C), lambda i: (0, i, 0)),
            pl.BlockSpec((K, _RB, C), lambda i: (0, i, 0)),
            pl.BlockSpec((_RB, C), lambda i: (i, 0)),
            pl.BlockSpec((DP, DP), lambda i: (0, 0)),
        ],
        out_specs=pl.BlockSpec((K, _RB, C), lambda i: (0, i, 0)),
        out_shape=jax.ShapeDtypeStruct((K, NTAB, C), jnp.float32),
    )(agg, sup, deg, w)


def _t6_body(agg_ref, sup_ref, deg_ref, out_ref):
    inv = 1.0 / (deg_ref[:, :1] + 1.0)
    for k in range(K):
        out_ref[k] = (agg_ref[k] + sup_ref[k]) * inv


def _t6(agg, sup, deg):
    return pl.pallas_call(
        _t6_body,
        grid=(_GRID,),
        in_specs=[
            pl.BlockSpec((K, _RB, C), lambda i: (0, i, 0)),
            pl.BlockSpec((K, _RB, C), lambda i: (0, i, 0)),
            pl.BlockSpec((_RB, C), lambda i: (i, 0)),
        ],
        out_specs=pl.BlockSpec((K, _RB, C), lambda i: (0, i, 0)),
        out_shape=jax.ShapeDtypeStruct((K, NBQ, C), jnp.float32),
    )(agg, sup, deg)


def _t8_body(sums_ref, cnt_ref, out_ref):
    inv = 1.0 / jnp.maximum(cnt_ref[:, :1], 1.0)
    vals = [sums_ref[k] * inv for k in range(K)]
    out_ref[...] = jnp.concatenate(vals, axis=1)


def _t8(sums, cnt):
    return pl.pallas_call(
        _t8_body,
        grid=(_GRID,),
        in_specs=[
            pl.BlockSpec((K, _RB, C), lambda i: (0, i, 0)),
            pl.BlockSpec((_RB, C), lambda i: (i, 0)),
        ],
        out_specs=pl.BlockSpec((_RB, DP), lambda i: (i, 0)),
        out_shape=jax.ShapeDtypeStruct((NTAB, DP), jnp.float32),
    )(sums, cnt)


@functools.partial(
    pl.kernel,
    compiler_params=_SC_PARAMS,
    out_type=(jax.ShapeDtypeStruct((NBQ, DEF), jnp.float32),
              jax.ShapeDtypeStruct((NBQ, DP), jnp.float32)),
    mesh=_MESH,
    scratch_types=[
        pltpu.VMEM((B,), jnp.int32),
        pltpu.VMEM((B,), jnp.int32),
        pltpu.VMEM((B, DEF), jnp.float32),
        pltpu.VMEM((B, DP), jnp.float32),
        pltpu.SemaphoreType.DMA,
        pltpu.SemaphoreType.DMA,
    ],
)
def _s1(bx_hbm, bgi_hbm, ef_hbm, rcw_hbm, ef_out, s1b_out,
        idx1, idx2, rows1, rows2, sem1, sem2):
    wid = lax.axis_index("s") * 2 + lax.axis_index("c")
    base0 = wid * RPT32

    @pl.loop(0, RPT32, step=B)
    def _(off):
        base = base0 + off
        pltpu.sync_copy(bx_hbm.at[pl.ds(base, B)], idx1)
        pltpu.sync_copy(bgi_hbm.at[pl.ds(base, B)], idx2)
        c1 = pltpu.async_copy(ef_hbm.at[idx1], rows1, sem1)
        c2 = pltpu.async_copy(rcw_hbm.at[idx2], rows2, sem2)
        c1.wait()
        c2.wait()
        pltpu.sync_copy(rows1, ef_out.at[pl.ds(base, B)])
        pltpu.sync_copy(rows2, s1b_out.at[pl.ds(base, B)])


@functools.partial(
    pl.kernel,
    compiler_params=_SC_PARAMS,
    out_type=(jax.ShapeDtypeStruct((NTAB, C), jnp.float32),
              jax.ShapeDtypeStruct((NTAB, C), jnp.float32)),
    mesh=_MESH,
    scratch_types=[
        pltpu.VMEM((B,), jnp.int32),
        pltpu.VMEM((B, C), jnp.float32),
        pltpu.VMEM_SHARED((NTAB, C), jnp.float32),
    ],
)
def _s3a(dst_hbm, bx_hbm, ones_hbm, zeros_hbm, deg_out, cnt_out,
         didx, ones_v, acc):
    cid = lax.axis_index("c")
    sid = lax.axis_index("s")
    stripe_slc = pl.ds(sid * STRIPE, STRIPE)
    pltpu.sync_copy(ones_hbm, ones_v)
    pltpu.sync_copy(zeros_hbm, acc.at[stripe_slc])
    plsc.subcore_barrier()

    @pl.when(cid == 0)
    def _():
        @pl.loop(0, EBT * B, step=B)
        def _(off):
            pltpu.sync_copy(dst_hbm.at[pl.ds(sid * (EBT * B) + off, B)], didx)
            pltpu.sync_copy(ones_v, acc.at[didx], add=True)

    @pl.when(cid == 1)
    def _():
        @pl.loop(0, RPT16, step=B)
        def _(off):
            pltpu.sync_copy(bx_hbm.at[pl.ds(sid * RPT16 + off, B)], didx)
            pltpu.sync_copy(ones_v, acc.at[didx], add=True)

    plsc.subcore_barrier()

    @pl.when(cid == 0)
    def _():
        pltpu.sync_copy(acc.at[stripe_slc], deg_out.at[stripe_slc])

    @pl.when(cid == 1)
    def _():
        pltpu.sync_copy(acc.at[stripe_slc], cnt_out.at[stripe_slc])


@functools.partial(
    pl.kernel,
    compiler_params=_SC_PARAMS,
    out_type=jax.ShapeDtypeStruct((K, NTAB, C), jnp.float32),
    mesh=_MESH,
    scratch_types=[
        pltpu.VMEM((SUP, B), jnp.int32),
        pltpu.VMEM((SUP, B), jnp.int32),
        pltpu.VMEM((SUP, B), jnp.int32),
        pltpu.VMEM((SUP, B), jnp.int32),
        pltpu.VMEM((SUP, B, C), jnp.float32),
        pltpu.VMEM((SUP, B, C), jnp.float32),
        pltpu.VMEM_SHARED((NTAB, C), jnp.float32),
        pltpu.SemaphoreType.DMA,
        pltpu.SemaphoreType.DMA,
        pltpu.SemaphoreType.DMA,
    ],
)
def _s3(src_hbm, dst_hbm, sup_hbm, zeros_hbm, agg_out,
        sidx0, sidx1, didx0, didx1, rows0, rows1, acc, gsem0, gsem1, ssem):
    cid = lax.axis_index("c")
    sid = lax.axis_index("s")
    stripe_slc = pl.ds(sid * STRIPE, STRIPE)
    bbase = sid * EBT
    sidx = (sidx0, sidx1)
    didx = (didx0, didx1)
    rows = (rows0, rows1)
    gsem = (gsem0, gsem1)

    def load_idx(s, p):
        blk = pl.ds(bbase + s * SUP, SUP)
        pltpu.sync_copy(src_hbm.at[blk], sidx[p])
        pltpu.sync_copy(dst_hbm.at[blk], didx[p])

    def fire_gathers(k, p):
        for j in range(SUP):
            pltpu.async_copy(sup_hbm.at[k].at[sidx[p].at[j]],
                             rows[p].at[j], gsem[p])

    def wait_gathers(k, p):
        for j in range(SUP):
            pltpu.make_async_copy(sup_hbm.at[k].at[sidx[p].at[j]],
                                  rows[p].at[j], gsem[p]).wait()

    def scatter_adds(p):
        descs = [pltpu.async_copy(rows[p].at[j], acc.at[didx[p].at[j]], ssem,
                                  add=True) for j in range(SUP)]
        for d in descs:
            d.wait()

    for k in range(K):
        @pl.when((k % 2) == cid)
        def _():
            pltpu.sync_copy(zeros_hbm, acc.at[stripe_slc])
            plsc.subcore_barrier()

            load_idx(0, 0)
            fire_gathers(k, 0)
            load_idx(1, 1)
            fire_gathers(k, 1)

            @pl.loop(0, NSUP, step=2)
            def _(s):
                i_last = (s + 2) >= NSUP
                wait_gathers(k, 0)
                scatter_adds(0)

                @pl.when(jnp.logical_not(i_last))
                def _():
                    load_idx(s + 2, 0)
                    fire_gathers(k, 0)

                wait_gathers(k, 1)
                scatter_adds(1)

                @pl.when(jnp.logical_not(i_last))
                def _():
                    load_idx(s + 3, 1)
                    fire_gathers(k, 1)

            plsc.subcore_barrier()
            pltpu.sync_copy(acc.at[stripe_slc], agg_out.at[k].at[stripe_slc])
            plsc.subcore_barrier()


@functools.partial(
    pl.kernel,
    compiler_params=_SC_PARAMS,
    out_type=jax.ShapeDtypeStruct((K, NTAB, C), jnp.float32),
    mesh=_MESH,
    scratch_types=[
        pltpu.VMEM((B,), jnp.int32),
        pltpu.VMEM((B,), jnp.int32),
        pltpu.VMEM((B, C), jnp.float32),
        pltpu.VMEM((B, C), jnp.float32),
        pltpu.VMEM_SHARED((NTAB, C), jnp.float32),
        pltpu.SemaphoreType.DMA,
        pltpu.SemaphoreType.DMA,
    ],
)
def _s7(bx_hbm, e_hbm, zeros_hbm, sums_out,
        didx0, didx1, rows0, rows1, acc, lsem0, lsem1):
    cid = lax.axis_index("c")
    sid = lax.axis_index("s")
    stripe_slc = pl.ds(sid * STRIPE, STRIPE)
    rbase = sid * RPT16
    nb = RPT16 // B
    didx = (didx0, didx1)
    rows = (rows0, rows1)
    lsem = (lsem0, lsem1)

    def fire_load(k, b, p):
        blk = pl.ds(rbase + b * B, B)
        pltpu.async_copy(bx_hbm.at[blk], didx[p], lsem[p])
        pltpu.async_copy(e_hbm.at[k].at[blk], rows[p], lsem[p])

    def wait_load(k, p):
        pltpu.make_async_copy(bx_hbm.at[pl.ds(0, B)], didx[p], lsem[p]).wait()
        pltpu.make_async_copy(e_hbm.at[k].at[pl.ds(0, B)], rows[p],
                              lsem[p]).wait()

    for k in range(K):
        @pl.when((k % 2) == cid)
        def _():
            pltpu.sync_copy(zeros_hbm, acc.at[stripe_slc])
            plsc.subcore_barrier()

            fire_load(k, 0, 0)

            @pl.loop(0, nb, step=2)
            def _(b):
                i_last = b + 2 >= nb
                fire_load(k, b + 1, 1)
                wait_load(k, 0)
                pltpu.sync_copy(rows[0], acc.at[didx[0]], add=True)

                @pl.when(jnp.logical_not(i_last))
                def _():
                    fire_load(k, b + 2, 0)

                wait_load(k, 1)
                pltpu.sync_copy(rows[1], acc.at[didx[1]], add=True)

            plsc.subcore_barrier()
            pltpu.sync_copy(acc.at[stripe_slc], sums_out.at[k].at[stripe_slc])
            plsc.subcore_barrier()


@functools.partial(
    pl.kernel,
    compiler_params=_SC_PARAMS,
    out_type=jax.ShapeDtypeStruct((NBQ, DP), jnp.float32),
    mesh=_MESH,
    scratch_types=[
        pltpu.VMEM((B,), jnp.int32),
        pltpu.VMEM((B, DP), jnp.float32),
        pltpu.SemaphoreType.DMA,
    ],
)
def _s9(bx_hbm, tab_hbm, z_out, idx, rows, sem):
    wid = lax.axis_index("s") * 2 + lax.axis_index("c")
    base0 = wid * RPT32

    @pl.loop(0, RPT32, step=B)
    def _(off):
        base = base0 + off
        pltpu.sync_copy(bx_hbm.at[pl.ds(base, B)], idx)
        pltpu.async_copy(tab_hbm.at[idx], rows, sem).wait()
        pltpu.sync_copy(rows, z_out.at[pl.ds(base, B)])


def kernel(entity_feat, relation_embeddings, W_rel_in, b_rel_in,
           W_gcn1, W_gcn2, b_x, b_node_graph_index, edge_index):
    f32 = jnp.float32
    efp = jnp.pad(entity_feat, ((0, NTAB - N), (0, DEF - D_FEAT)))
    w1p = jnp.pad(W_gcn1, ((0, 0), (0, DP - D_HID)))
    w1top = jnp.pad(w1p[:D_FEAT], ((0, DEF - D_FEAT), (0, 0)))
    w1bot = w1p[D_FEAT:D_HID]
    w2p = jnp.pad(W_gcn2, ((0, DP - D_HID), (0, DP - D_HID)))
    pad_bx = jnp.full((NBQ - NB,), DUMMY, jnp.int32)
    bxp = jnp.concatenate([b_x.astype(jnp.int32), pad_bx])
    bgip = jnp.concatenate([b_node_graph_index.astype(jnp.int32),
                            jnp.zeros((NBQ - NB,), jnp.int32)])
    pad_e = jnp.full((EP - E,), DUMMY, jnp.int32)
    srcp = jnp.concatenate([edge_index[0].astype(jnp.int32), pad_e])
    dstp = jnp.concatenate([edge_index[1].astype(jnp.int32), pad_e])
    srcb = srcp.reshape(EP // B, B)
    dstb = dstp.reshape(EP // B, B)
    zeros_hbm = jnp.zeros((STRIPE, C), f32)
    ones_hbm = jnp.ones((B, C), f32)
    brel = b_rel_in.reshape(1, D_FEAT)

    rcw = _t0(relation_embeddings, W_rel_in, brel, w1bot)
    ef_g, s1b = _s1(bxp, bgip, efp, rcw)
    deg, cnt = _s3a(dstp, bxp, ones_hbm, zeros_hbm)
    sup1 = _t2(ef_g, s1b, w1top)
    agg1 = _s3(srcb, dstb, sup1, zeros_hbm)
    sup2 = _t4(agg1, sup1, deg, w2p)
    agg2 = _s3(srcb, dstb, sup2, zeros_hbm)
    e = _t6(agg2, sup2, deg)
    sums = _s7(bxp, e, zeros_hbm)
    out_tab = _t8(sums, cnt)
    zf = _s9(bxp, out_tab)
    return zf[:NB, :D_HID]
